# baseline (speedup 1.0000x reference)
import functools

import jax
import jax.numpy as jnp
from jax import lax
from jax.experimental import pallas as pl
from jax.experimental.pallas import tpu as pltpu
from jax.experimental.pallas import tpu_sc as plsc

_B, _N, _D, _C, _Q = 4, 8192, 256, 91, 300
_BLK = 1024


def _mlp_body(x_ref, anch_ref, ws1, bs1, ws2, bs2, ws3, bs3, wb1, bb1, wb2, bb2,
              logits_ref, boxes_ref, scores_ref, inv_ref):
    i = pl.program_id(0)

    @pl.when(i == 0)
    def _():
        a = jnp.clip(anch_ref[...], 1e-06, 1 - 1e-06)
        inv_ref[...] = jnp.log(a / (1.0 - a))

    x = x_ref[...]
    g = jnp.maximum(jnp.dot(x, wb1[...], preferred_element_type=jnp.float32) + bb1[...], 0.0)
    h = jnp.maximum(jnp.dot(x, ws1[...], preferred_element_type=jnp.float32) + bs1[...], 0.0)
    delta = jnp.dot(g, wb2[...], preferred_element_type=jnp.float32) + bb2[...]
    h = jnp.maximum(jnp.dot(h, ws2[...], preferred_element_type=jnp.float32) + bs2[...], 0.0)
    logits = jnp.dot(h, ws3[...], preferred_element_type=jnp.float32) + bs3[...]
    logits_ref[...] = logits
    nanch = _N // _BLK
    inv = inv_ref[pl.ds((i % nanch) * _BLK, _BLK), :]
    boxes_ref[...] = jax.nn.sigmoid(inv + delta)
    lt = logits.T
    m = jnp.max(lt, axis=0)
    scores_ref[...] = jax.nn.sigmoid(m)[None, None, :]


def _run_mlp(x, anchors, ws1, bs1, ws2, bs2, ws3, bs3, wb1, bb1, wb2, bb2):
    bn = _B * _N
    nb = bn // _BLK
    full = lambda arr: pl.BlockSpec(arr.shape, lambda i: (0,) * arr.ndim)
    grid_spec = pltpu.PrefetchScalarGridSpec(
        num_scalar_prefetch=0,
        grid=(nb,),
        scratch_shapes=[pltpu.VMEM((_N, 4), jnp.float32)],
        in_specs=[
            pl.BlockSpec((_BLK, _D), lambda i: (i, 0)),
            full(anchors),
            full(ws1), full(bs1), full(ws2), full(bs2), full(ws3), full(bs3),
            full(wb1), full(bb1), full(wb2), full(bb2),
        ],
        out_specs=[
            pl.BlockSpec((_BLK, _C), lambda i: (i, 0)),
            pl.BlockSpec((_BLK, 4), lambda i: (i, 0)),
            pl.BlockSpec((1, 1, _BLK), lambda i: (i, 0, 0)),
        ],
    )
    return pl.pallas_call(
        _mlp_body,
        grid_spec=grid_spec,
        out_shape=[
            jax.ShapeDtypeStruct((bn, _C), jnp.float32),
            jax.ShapeDtypeStruct((bn, 4), jnp.float32),
            jax.ShapeDtypeStruct((nb, 1, _BLK), jnp.float32),
        ],
        compiler_params=pltpu.CompilerParams(
            dimension_semantics=("arbitrary",),
        ),
    )(x, anchors, ws1, bs1, ws2, bs2, ws3, bs3, wb1, bb1, wb2, bb2)


_L = 16
_SURV = 384
_SBUF = 416
_WPB = 8
_RSL = 48
_GCH = 40
_QP = 304


def _iota16():
    return lax.iota(jnp.int32, _L)


def _sget(vec, i):
    return jnp.sum(jnp.where(_iota16() == i, vec, 0))


def _sc_body(skey_hbm, feats_hbm, boxes_hbm, qc_hbm, qr_hbm,
             keys_v, hist_v, skc_v, sgc_v, tgc_v, sk_all, sg_all,
             st_idx0, st_idx1, st_idx2, st_val0, st_val1, st_val2,
             mr_idx, mr_val, meta_v, z_v,
             gidx_v, rows_v, idx4a, idx4b, brows_a, brows_b,
             sh_key, sh_gidx, sh_meta, sh_out, sem):
    c = lax.axis_index("c")
    s = lax.axis_index("s")
    b = 2 * c + s // _WPB
    w = s % _WPB
    slot = s // _WPB
    iot = _iota16()

    @pl.when(w == 0)
    def _leader():
        pltpu.sync_copy(skey_hbm.at[pl.ds(b * _N, _N)], keys_v)
        for j in range(_SURV // _L):
            z_v[pl.ds(j * _L, _L)] = jnp.zeros((_L,), jnp.int32)
        pltpu.sync_copy(z_v, sh_out.at[pl.ds(slot * _SURV, _SURV)])

        base = jnp.int32(0)
        need = jnp.int32(_Q)
        for p in range(4):
            sh_d = 24 - 8 * p

            def zero_body(j, _):
                hist_v[pl.ds(j * _L, _L)] = jnp.zeros((_L,), jnp.int32)
                return 0
            lax.fori_loop(0, 256, zero_body, 0)

            def fill_body(j, _, base=base, p=p, sh_d=sh_d):
                kv = keys_v[pl.ds(j * _L, _L)]
                digit = (kv >> sh_d) & 0xFF
                idx = digit * _L + iot
                if p == 0:
                    plsc.addupdate_scatter(hist_v, [idx], jnp.ones((_L,), jnp.int32))
                else:
                    elig = (kv >> (sh_d + 8)) == (base >> (sh_d + 8))
                    cnt = jnp.sum(jnp.where(elig, 1, 0))

                    @pl.when(cnt > 0)
                    def _():
                        plsc.addupdate_scatter(
                            hist_v, [idx], jnp.where(elig, 1, 0).astype(jnp.int32))
                return 0
            lax.fori_loop(0, _N // _L, fill_body, 0)

            def grp_body(g, gt_vec):
                def row_body(d, acc):
                    return acc + hist_v[pl.ds((g * _L + d) * _L, _L)]
                acc = lax.fori_loop(0, _L, row_body, jnp.zeros((_L,), jnp.int32))
                return jnp.where(iot == g, jnp.sum(acc), gt_vec)
            gt_vec = lax.fori_loop(0, _L, grp_body, jnp.zeros((_L,), jnp.int32))
            sfx_g = lax.rev(jnp.cumsum(lax.rev(gt_vec, (0,))), (0,))
            gstar = jnp.max(jnp.where(sfx_g >= need, iot, -1))
            need_g = need - (_sget(sfx_g, gstar) - _sget(gt_vec, gstar))

            def dt_body(d, dt_vec):
                row = hist_v[pl.ds((gstar * _L + d) * _L, _L)]
                return jnp.where(iot == d, jnp.sum(row), dt_vec)
            dt_vec = lax.fori_loop(0, _L, dt_body, jnp.zeros((_L,), jnp.int32))
            sfx_d = lax.rev(jnp.cumsum(lax.rev(dt_vec, (0,))), (0,))
            dstar = jnp.max(jnp.where(sfx_d >= need_g, iot, -1))
            need = need_g - (_sget(sfx_d, dstar) - _sget(dt_vec, dstar))
            base = base | ((gstar * _L + dstar) << sh_d)

        thr = base

        def coll_body(j, carry):
            n_gt, n_eq = carry
            kv = keys_v[pl.ds(j * _L, _L)]
            gv = jnp.full((_L,), b * _N + j * _L, jnp.int32) + iot
            m_gt = kv > thr
            c1 = jnp.sum(jnp.where(m_gt, 1, 0))

            @pl.when(c1 > 0)
            def _():
                pos = n_gt + jnp.cumsum(jnp.where(m_gt, 1, 0)) - 1
                pos = jnp.where(m_gt, pos, _SBUF - 1)
                plsc.store_scatter(skc_v, [pos], kv, mask=m_gt)
                plsc.store_scatter(sgc_v, [pos], gv, mask=m_gt)
            m_eq = kv == thr
            c2 = jnp.sum(jnp.where(m_eq, 1, 0))

            @pl.when((c2 > 0) & (n_eq < _Q + 20))
            def _():
                pos = n_eq + jnp.cumsum(jnp.where(m_eq, 1, 0)) - 1
                pos = jnp.where(m_eq, pos, _SBUF - 1)
                plsc.store_scatter(tgc_v, [pos], gv, mask=m_eq)
            return (n_gt + c1, n_eq + c2)
        n_gt, _n_eq = lax.fori_loop(0, _N // _L, coll_body,
                                    (jnp.int32(0), jnp.int32(0)))

        meta_v[...] = jnp.where(iot == 0, n_gt, 0) + jnp.where(iot == 1, need, 0)
        pltpu.sync_copy(meta_v, sh_meta.at[pl.ds(slot * _L, _L)])
        pltpu.sync_copy(skc_v.at[pl.ds(0, _SURV)], sh_key.at[pl.ds(slot * _SBUF, _SURV)])
        pltpu.sync_copy(sgc_v.at[pl.ds(0, _SURV)], sh_gidx.at[pl.ds(slot * _SBUF, _SURV)])

        dump = slot * _SURV + _SURV - _L + iot
        st_idx = (st_idx0, st_idx1, st_idx2)
        st_val = (st_val0, st_val1, st_val2)
        for t in range(_SURV // _L):
            posv = jnp.full((_L,), t * _L, jnp.int32) + iot
            r = jnp.where(posv < need, n_gt + posv + slot * _SURV, dump)
            st_idx[t // 8][pl.ds((t % 8) * _L, _L)] = r
            st_val[t // 8][pl.ds((t % 8) * _L, _L)] = tgc_v[pl.ds(t * _L, _L)]
        for j in range(3):
            pltpu.sync_copy(st_val[j], sh_out.at[st_idx[j]], add=True)

    plsc.subcore_barrier()

    pltpu.sync_copy(sh_meta.at[pl.ds(slot * _L, _L)], meta_v)
    c_gt = jnp.sum(jnp.where(iot == 0, meta_v[...], 0))
    pltpu.sync_copy(sh_key.at[pl.ds(slot * _SBUF, _SURV)], sk_all)
    pltpu.sync_copy(sh_gidx.at[pl.ds(slot * _SBUF, _SURV)], sg_all)
    nch = (c_gt + _L - 1) // _L
    dump = slot * _SURV + _SURV - _L + iot

    def chunk_body(t, _):
        cbase = w * _RSL + t * _L
        kv = sk_all[pl.ds(cbase, _L)]
        gv = sg_all[pl.ds(cbase, _L)]

        @pl.when(cbase < c_gt)
        def _():
            def elem_body(e, rvec):
                ke = _sget(kv, e)
                ge = _sget(gv, e)

                def cmp_body(j, acc):
                    sk = sk_all[pl.ds(j * _L, _L)]
                    sg = sg_all[pl.ds(j * _L, _L)]
                    posj = jnp.full((_L,), j * _L, jnp.int32) + iot
                    hit = (posj < c_gt) & ((sk > ke) | ((sk == ke) & (sg < ge)))
                    return acc + jnp.where(hit, 1, 0)
                acc = lax.fori_loop(0, nch, cmp_body, jnp.zeros((_L,), jnp.int32))
                return jnp.where(iot == e, jnp.sum(acc), rvec)
            rvec = lax.fori_loop(0, _L, elem_body, jnp.zeros((_L,), jnp.int32))
            posv = jnp.full((_L,), cbase, jnp.int32) + iot
            mr_idx[pl.ds(t * _L, _L)] = jnp.where(
                posv < c_gt, rvec + slot * _SURV, dump)
            mr_val[pl.ds(t * _L, _L)] = gv

        @pl.when(cbase >= c_gt)
        def _():
            mr_idx[pl.ds(t * _L, _L)] = dump
            mr_val[pl.ds(t * _L, _L)] = jnp.zeros((_L,), jnp.int32)
        return 0
    lax.fori_loop(0, _RSL // _L, chunk_body, 0)
    pltpu.sync_copy(mr_val, sh_out.at[mr_idx], add=True)
    plsc.subcore_barrier()

    qbase = b * _QP + w * _GCH
    lbase = slot * _SURV + w * _GCH
    pltpu.sync_copy(sh_out.at[pl.ds(lbase, _GCH)], gidx_v)
    pltpu.async_copy(feats_hbm.at[gidx_v], rows_v, sem).wait()
    for t in range(10):
        p = jnp.full((_L,), t * _L, jnp.int32) + iot
        rowvals = plsc.load_gather(gidx_v, [p >> 2])
        ev = rowvals * 4 + (p & 3)
        if t < 5:
            idx4a[pl.ds(t * _L, _L)] = ev
        else:
            idx4b[pl.ds((t - 5) * _L, _L)] = ev
    pltpu.async_copy(boxes_hbm.at[idx4a], brows_a, sem).wait()
    pltpu.async_copy(boxes_hbm.at[idx4b], brows_b, sem).wait()
    qb4 = qbase * 4

    @pl.when(w < _WPB - 1)
    def _():
        pltpu.sync_copy(rows_v, qc_hbm.at[pl.ds(qbase, _GCH)])
        pltpu.sync_copy(brows_a, qr_hbm.at[pl.ds(qb4, 80)])
        pltpu.sync_copy(brows_b, qr_hbm.at[pl.ds(qb4 + 80, 80)])

    @pl.when(w == _WPB - 1)
    def _():
        last = _QP - (_WPB - 1) * _GCH
        pltpu.sync_copy(rows_v.at[pl.ds(0, last)], qc_hbm.at[pl.ds(qbase, last)])
        pltpu.sync_copy(brows_a, qr_hbm.at[pl.ds(qb4, 80)])
        pltpu.sync_copy(brows_b.at[pl.ds(0, last * 4 - 80)],
                        qr_hbm.at[pl.ds(qb4 + 80, last * 4 - 80)])


def _run_sc_topk(skey_flat, feats2d, boxes2d):
    mesh = plsc.VectorSubcoreMesh(core_axis_name="c", subcore_axis_name="s")
    fn = pl.kernel(
        _sc_body,
        mesh=mesh,
        out_type=[
            jax.ShapeDtypeStruct((_B * _QP, _D), jnp.float32),
            jax.ShapeDtypeStruct((_B * _QP * 4,), jnp.float32),
        ],
        scratch_types=[
            pltpu.VMEM((_N,), jnp.int32),
            pltpu.VMEM((4096,), jnp.int32),
            pltpu.VMEM((_SBUF,), jnp.int32),
            pltpu.VMEM((_SBUF,), jnp.int32),
            pltpu.VMEM((_SBUF,), jnp.int32),
            pltpu.VMEM((_SURV,), jnp.int32),
            pltpu.VMEM((_SURV,), jnp.int32),
            pltpu.VMEM((128,), jnp.int32),
            pltpu.VMEM((128,), jnp.int32),
            pltpu.VMEM((128,), jnp.int32),
            pltpu.VMEM((128,), jnp.int32),
            pltpu.VMEM((128,), jnp.int32),
            pltpu.VMEM((128,), jnp.int32),
            pltpu.VMEM((_RSL,), jnp.int32),
            pltpu.VMEM((_RSL,), jnp.int32),
            pltpu.VMEM((_L,), jnp.int32),
            pltpu.VMEM((_SURV,), jnp.int32),
            pltpu.VMEM((_GCH,), jnp.int32),
            pltpu.VMEM((_GCH, _D), jnp.float32),
            pltpu.VMEM((80,), jnp.int32),
            pltpu.VMEM((80,), jnp.int32),
            pltpu.VMEM((80,), jnp.float32),
            pltpu.VMEM((80,), jnp.float32),
            pltpu.VMEM_SHARED((2 * _SBUF,), jnp.int32),
            pltpu.VMEM_SHARED((2 * _SBUF,), jnp.int32),
            pltpu.VMEM_SHARED((2 * _L,), jnp.int32),
            pltpu.VMEM_SHARED((2 * _SURV,), jnp.int32),
            pltpu.SemaphoreType.DMA,
        ],
        compiler_params=pltpu.CompilerParams(needs_layout_passes=False),
    )
    return fn(skey_flat, feats2d, boxes2d)


def kernel(flat_feats, flat_anchors, Ws1, bs1, Ws2, bs2, Ws3, bs3, Wb1, bb1, Wb2, bb2):
    bn = _B * _N
    x = flat_feats.reshape(bn, _D)
    logits, boxes, scores = _run_mlp(
        x, flat_anchors,
        Ws1, bs1.reshape(1, -1), Ws2, bs2.reshape(1, -1), Ws3, bs3.reshape(1, -1),
        Wb1, bb1.reshape(1, -1), Wb2, bb2.reshape(1, -1))
    enc_logits = logits.reshape(_B, _N, _C)
    enc_boxes = boxes.reshape(_B, _N, 4)
    s = scores.reshape(_B, _N)
    skey_flat = jax.lax.bitcast_convert_type(s, jnp.int32).reshape(bn)
    qc, qr = _run_sc_topk(skey_flat, x, boxes.reshape(-1))
    query_content = qc.reshape(_B, _QP, _D)[:, :_Q]
    query_ref_pts = qr.reshape(_B, _QP, 4)[:, :_Q]
    return (query_content, query_ref_pts, enc_logits, enc_boxes)
